# Initial kernel scaffold; baseline (speedup 1.0000x reference)
import functools

import jax
import jax.numpy as jnp
from jax import lax
from jax.experimental import pallas as pl
from jax.experimental.pallas import tpu as pltpu
from jax.experimental.pallas import tpu_sc as plsc

N = 10000
E = 320000
D = 128
H = 1024
G = 16
O = 4

NC = 2
NS = 16
NW = NC * NS
EPW = E // NW
CHUNK = 80
NCHUNK = EPW // CHUNK
RPT = N // NS
ZR = 125
CW = 16


def _sc_body(src_hbm, dst_hbm, x_hbm, s_out, c_out,
             idxs_v, idxd_v, rows_v, ones_v, zrow_v, zcnt_v,
             s_sh, c_sh, sem):
    cid = lax.axis_index("c")
    sid = lax.axis_index("s")
    w = sid * NC + cid

    zeros16 = jnp.zeros((16,), jnp.float32)
    ones16 = jnp.ones((16,), jnp.float32)

    def fill_ones(i, carry):
        ones_v[i, :] = ones16
        return carry

    lax.fori_loop(0, CHUNK, fill_ones, 0)

    def fill_zrow(i, carry):
        for j in range(D // 16):
            zrow_v[i, pl.ds(j * 16, 16)] = zeros16
        return carry

    lax.fori_loop(0, ZR, fill_zrow, 0)

    def fill_zcnt(i, carry):
        zcnt_v[i, :] = zeros16
        return carry

    lax.fori_loop(0, ZR, fill_zcnt, 0)

    r0 = sid * RPT
    for k in range(RPT // ZR):
        pltpu.sync_copy(zrow_v, s_sh.at[pl.ds(r0 + k * ZR, ZR)])
        pltpu.sync_copy(zcnt_v, c_sh.at[pl.ds(r0 + k * ZR, ZR)])
    plsc.subcore_barrier()

    e0 = w * EPW

    def chunk(c, carry):
        off = e0 + c * CHUNK
        pltpu.sync_copy(src_hbm.at[pl.ds(off, CHUNK)], idxs_v)
        pltpu.sync_copy(dst_hbm.at[pl.ds(off, CHUNK)], idxd_v)
        pltpu.async_copy(x_hbm.at[idxs_v], rows_v, sem).wait()
        pltpu.sync_copy(rows_v, s_sh.at[idxd_v], add=True)
        pltpu.sync_copy(ones_v, c_sh.at[idxd_v], add=True)
        return carry

    lax.fori_loop(0, NCHUNK, chunk, 0)
    plsc.subcore_barrier()

    pltpu.sync_copy(s_sh.at[pl.ds(r0, RPT)], s_out.at[cid, pl.ds(r0, RPT)])
    pltpu.sync_copy(c_sh.at[pl.ds(r0, RPT)], c_out.at[cid, pl.ds(r0, RPT)])


@jax.jit
def _sc_segment_sum(src, dst, x):
    mesh = plsc.VectorSubcoreMesh(core_axis_name="c", subcore_axis_name="s")
    return pl.kernel(
        _sc_body,
        out_type=[
            jax.ShapeDtypeStruct((NC, N, D), jnp.float32),
            jax.ShapeDtypeStruct((NC, N, CW), jnp.float32),
        ],
        mesh=mesh,
        scratch_types=[
            pltpu.VMEM((CHUNK,), jnp.int32),
            pltpu.VMEM((CHUNK,), jnp.int32),
            pltpu.VMEM((CHUNK, D), jnp.float32),
            pltpu.VMEM((CHUNK, CW), jnp.float32),
            pltpu.VMEM((ZR, D), jnp.float32),
            pltpu.VMEM((ZR, CW), jnp.float32),
            pltpu.VMEM_SHARED((N, D), jnp.float32),
            pltpu.VMEM_SHARED((N, CW), jnp.float32),
            pltpu.SemaphoreType.DMA,
        ],
    )(src, dst, x)


BR = 400
NB = N // BR


def _tc_body(x_ref, s0_ref, s1_ref, c0_ref, c1_ref, bat_ref,
             root_ref, w_ref, b1_ref, wfc_ref, bfc_ref, out_ref, acc):
    i = pl.program_id(0)
    xa = x_ref[...]
    sa = s0_ref[...] + s1_ref[...]
    cnt = c0_ref[:, 0:1] + c1_ref[:, 0:1]
    h = sa / jnp.maximum(cnt, 1.0)
    o = (jnp.dot(xa, root_ref[...], preferred_element_type=jnp.float32)
         + jnp.dot(h, w_ref[...], preferred_element_type=jnp.float32)
         + b1_ref[...])
    o = jnp.maximum(o, 0.0)
    bb = bat_ref[0, 0, :]
    onehot = (lax.broadcasted_iota(jnp.int32, (G, BR), 0)
              == bb[None, :]).astype(jnp.float32)
    part = jnp.dot(onehot, o, preferred_element_type=jnp.float32)

    @pl.when(i == 0)
    def _():
        acc[...] = jnp.zeros_like(acc)

    acc[...] += part

    @pl.when(i == NB - 1)
    def _():
        out_ref[...] = (jnp.dot(acc[...], wfc_ref[...],
                                preferred_element_type=jnp.float32)
                        + bfc_ref[...])


@jax.jit
def _tc_dense(x, s0, s1, c0, c1, bat3, root, W0, b1r, Wfc, bfcr):
    return pl.pallas_call(
        _tc_body,
        grid=(NB,),
        in_specs=[
            pl.BlockSpec((BR, D), lambda i: (i, 0)),
            pl.BlockSpec((BR, D), lambda i: (i, 0)),
            pl.BlockSpec((BR, D), lambda i: (i, 0)),
            pl.BlockSpec((BR, CW), lambda i: (i, 0)),
            pl.BlockSpec((BR, CW), lambda i: (i, 0)),
            pl.BlockSpec((1, 1, BR), lambda i: (i, 0, 0)),
            pl.BlockSpec((D, H), lambda i: (0, 0)),
            pl.BlockSpec((D, H), lambda i: (0, 0)),
            pl.BlockSpec((1, H), lambda i: (0, 0)),
            pl.BlockSpec((H, O), lambda i: (0, 0)),
            pl.BlockSpec((1, O), lambda i: (0, 0)),
        ],
        out_specs=pl.BlockSpec((G, O), lambda i: (0, 0)),
        out_shape=jax.ShapeDtypeStruct((G, O), jnp.float32),
        scratch_shapes=[pltpu.VMEM((G, H), jnp.float32)],
        compiler_params=pltpu.CompilerParams(
            dimension_semantics=("arbitrary",),
        ),
    )(x, s0, s1, c0, c1, bat3, root, W0, b1r, Wfc, bfcr)


def kernel(x, edge_index, edge_attr, batch, W, root, b1, Wfc, bfc):
    src = edge_index[0]
    dst = edge_index[1]
    s_part, c_part = _sc_segment_sum(src, dst, x)
    bat3 = batch.reshape(NB, 1, BR)
    return _tc_dense(x, s_part[0], s_part[1], c_part[0], c_part[1], bat3,
                     root, W[0], b1.reshape(1, H), Wfc, bfc.reshape(1, O))


# speedup vs baseline: 5.8397x; 5.8397x over previous
import functools

import jax
import jax.numpy as jnp
from jax import lax
from jax.experimental import pallas as pl
from jax.experimental.pallas import tpu as pltpu
from jax.experimental.pallas import tpu_sc as plsc

N = 10000
E = 320000
D = 128
H = 1024
G = 16
O = 4

NC = 2
NS = 16
NW = NC * NS
EPW = E // NW
CHUNK = 80
NCHUNK = EPW // CHUNK
RPT = N // NS
ZR = 125
CW = 16


def _sc_body(src_hbm, dst_hbm, x_hbm, s_out, c_out,
             idxs_v, idxd_v, rows_v, ones_v, zrow_v, zcnt_v,
             s_sh, c_sh, sem):
    cid = lax.axis_index("c")
    sid = lax.axis_index("s")
    w = sid * NC + cid

    zeros16 = jnp.zeros((16,), jnp.float32)
    ones16 = jnp.ones((16,), jnp.float32)

    def fill_ones(i, carry):
        ones_v[i, :] = ones16
        return carry

    lax.fori_loop(0, CHUNK, fill_ones, 0)

    def fill_zrow(i, carry):
        for j in range(D // 16):
            zrow_v[i, pl.ds(j * 16, 16)] = zeros16
        return carry

    lax.fori_loop(0, ZR, fill_zrow, 0)

    def fill_zcnt(i, carry):
        zcnt_v[i, :] = zeros16
        return carry

    lax.fori_loop(0, ZR, fill_zcnt, 0)

    r0 = sid * RPT
    for k in range(RPT // ZR):
        pltpu.sync_copy(zrow_v, s_sh.at[pl.ds(r0 + k * ZR, ZR)])
        pltpu.sync_copy(zcnt_v, c_sh.at[pl.ds(r0 + k * ZR, ZR)])
    plsc.subcore_barrier()

    e0 = w * EPW

    def chunk(c, carry):
        off = e0 + c * CHUNK
        pltpu.sync_copy(src_hbm.at[pl.ds(off, CHUNK)], idxs_v)
        pltpu.sync_copy(dst_hbm.at[pl.ds(off, CHUNK)], idxd_v)
        pltpu.async_copy(x_hbm.at[idxs_v], rows_v, sem).wait()
        pltpu.sync_copy(rows_v, s_sh.at[idxd_v], add=True)
        pltpu.sync_copy(ones_v, c_sh.at[idxd_v], add=True)
        return carry

    lax.fori_loop(0, NCHUNK, chunk, 0)
    plsc.subcore_barrier()

    pltpu.sync_copy(s_sh.at[pl.ds(r0, RPT)], s_out.at[cid, pl.ds(r0, RPT)])
    pltpu.sync_copy(c_sh.at[pl.ds(r0, RPT)], c_out.at[cid, pl.ds(r0, RPT)])


@jax.jit
def _sc_segment_sum(src, dst, x):
    mesh = plsc.VectorSubcoreMesh(core_axis_name="c", subcore_axis_name="s")
    return pl.kernel(
        _sc_body,
        out_type=[
            jax.ShapeDtypeStruct((NC, N, D), jnp.float32),
            jax.ShapeDtypeStruct((NC, N, CW), jnp.float32),
        ],
        mesh=mesh,
        scratch_types=[
            pltpu.VMEM((CHUNK,), jnp.int32),
            pltpu.VMEM((CHUNK,), jnp.int32),
            pltpu.VMEM((CHUNK, D), jnp.float32),
            pltpu.VMEM((CHUNK, CW), jnp.float32),
            pltpu.VMEM((ZR, D), jnp.float32),
            pltpu.VMEM((ZR, CW), jnp.float32),
            pltpu.VMEM_SHARED((N, D), jnp.float32),
            pltpu.VMEM_SHARED((N, CW), jnp.float32),
            pltpu.SemaphoreType.DMA,
        ],
        compiler_params=pltpu.CompilerParams(use_tc_tiling_on_sc=False),
    )(src, dst, x)


BR = 400
NB = N // BR


def _tc_body(x_ref, s0_ref, s1_ref, c0_ref, c1_ref, bat_ref,
             root_ref, w_ref, b1_ref, wfc_ref, bfc_ref, out_ref, acc):
    i = pl.program_id(0)
    xa = x_ref[...]
    sa = s0_ref[...] + s1_ref[...]
    cnt = c0_ref[:, 0:1] + c1_ref[:, 0:1]
    h = sa / jnp.maximum(cnt, 1.0)
    o = (jnp.dot(xa, root_ref[...], preferred_element_type=jnp.float32)
         + jnp.dot(h, w_ref[...], preferred_element_type=jnp.float32)
         + b1_ref[...])
    o = jnp.maximum(o, 0.0)
    bb = bat_ref[0, 0, :]
    onehot = (lax.broadcasted_iota(jnp.int32, (G, BR), 0)
              == bb[None, :]).astype(jnp.float32)
    part = jnp.dot(onehot, o, preferred_element_type=jnp.float32)

    @pl.when(i == 0)
    def _():
        acc[...] = jnp.zeros_like(acc)

    acc[...] += part

    @pl.when(i == NB - 1)
    def _():
        out_ref[...] = (jnp.dot(acc[...], wfc_ref[...],
                                preferred_element_type=jnp.float32)
                        + bfc_ref[...])


@jax.jit
def _tc_dense(x, s0, s1, c0, c1, bat3, root, W0, b1r, Wfc, bfcr):
    return pl.pallas_call(
        _tc_body,
        grid=(NB,),
        in_specs=[
            pl.BlockSpec((BR, D), lambda i: (i, 0)),
            pl.BlockSpec((BR, D), lambda i: (i, 0)),
            pl.BlockSpec((BR, D), lambda i: (i, 0)),
            pl.BlockSpec((BR, CW), lambda i: (i, 0)),
            pl.BlockSpec((BR, CW), lambda i: (i, 0)),
            pl.BlockSpec((1, 1, BR), lambda i: (i, 0, 0)),
            pl.BlockSpec((D, H), lambda i: (0, 0)),
            pl.BlockSpec((D, H), lambda i: (0, 0)),
            pl.BlockSpec((1, H), lambda i: (0, 0)),
            pl.BlockSpec((H, O), lambda i: (0, 0)),
            pl.BlockSpec((1, O), lambda i: (0, 0)),
        ],
        out_specs=pl.BlockSpec((G, O), lambda i: (0, 0)),
        out_shape=jax.ShapeDtypeStruct((G, O), jnp.float32),
        scratch_shapes=[pltpu.VMEM((G, H), jnp.float32)],
        compiler_params=pltpu.CompilerParams(
            dimension_semantics=("arbitrary",),
        ),
    )(x, s0, s1, c0, c1, bat3, root, W0, b1r, Wfc, bfcr)


def kernel(x, edge_index, edge_attr, batch, W, root, b1, Wfc, bfc):
    src = edge_index[0]
    dst = edge_index[1]
    s_part, c_part = _sc_segment_sum(src, dst, x)
    bat3 = batch.reshape(NB, 1, BR)
    return _tc_dense(x, s_part[0], s_part[1], c_part[0], c_part[1], bat3,
                     root, W[0], b1.reshape(1, H), Wfc, bfc.reshape(1, O))
